# Initial kernel scaffold; baseline (speedup 1.0000x reference)
import jax
import jax.numpy as jnp
from jax.experimental import pallas as pl


def kernel(X, edge_index, edge_weight, adj_label, eps, W1, W_mean, W_logsig):
    raise NotImplementedError("write your pallas kernel here")


# speedup vs baseline: 1.3769x; 1.3769x over previous
import functools

import jax
import jax.numpy as jnp
from jax.experimental import pallas as pl
from jax.experimental.pallas import tpu as pltpu

N = 4096
E = 131072
N_X = 512
N_H = 256
N_Z = 64
POS_WEIGHT = float(N * N - E) / E
NORM_LOSS = (N * N) / float((N * N - E) * 2)


def _matmul_body(x_ref, w_ref, o_ref):
    o_ref[...] = jnp.dot(x_ref[...], w_ref[...],
                         preferred_element_type=jnp.float32)


def _matmul(x, w, blk_rows):
    m, k = x.shape
    _, n = w.shape
    return pl.pallas_call(
        _matmul_body,
        grid=(m // blk_rows,),
        in_specs=[
            pl.BlockSpec((blk_rows, k), lambda i: (i, 0)),
            pl.BlockSpec((k, n), lambda i: (0, 0)),
        ],
        out_specs=pl.BlockSpec((blk_rows, n), lambda i: (i, 0)),
        out_shape=jax.ShapeDtypeStruct((m, n), jnp.float32),
    )(x, w)


def _reparam_body(zm_ref, zls_ref, eps_ref, z_ref, kl_ref):
    zm = zm_ref[...]
    zls = zls_ref[...]
    sig = jnp.exp(zls)
    z_ref[...] = zm + eps_ref[...] * sig
    kl_sum = jnp.sum(1.0 + 2.0 * zls - zm * zm - sig * sig)
    kl_ref[0, 0] = (-0.5 / (N * N)) * kl_sum


def _reparam_kl(zm, zls, eps):
    return pl.pallas_call(
        _reparam_body,
        out_specs=(
            pl.BlockSpec((N, N_Z), lambda: (0, 0)),
            pl.BlockSpec(memory_space=pltpu.SMEM),
        ),
        out_shape=(
            jax.ShapeDtypeStruct((N, N_Z), jnp.float32),
            jax.ShapeDtypeStruct((1, 1), jnp.float32),
        ),
    )(zm, zls, eps)


def _decoder_body(z_ref, adj_ref, a_ref, loss_ref, *, blk):
    i = pl.program_id(0)
    z_blk = z_ref[pl.ds(i * blk, blk), :]
    logits = jax.lax.dot_general(
        z_blk, z_ref[...], (((1,), (1,)), ((), ())),
        preferred_element_type=jnp.float32)
    a_ref[...] = logits
    labels = adj_ref[...]
    log_weight = 1.0 + (POS_WEIGHT - 1.0) * labels
    ce = (1.0 - labels) * logits + log_weight * (
        jnp.log1p(jnp.exp(-jnp.abs(logits))) + jnp.maximum(-logits, 0.0))
    part = jnp.sum(ce) * (NORM_LOSS / (N * N))

    @pl.when(i == 0)
    def _():
        loss_ref[0, 0] = 0.0

    loss_ref[0, 0] += part


def _decoder_loss(z, adj, blk_rows=256):
    body = functools.partial(_decoder_body, blk=blk_rows)
    return pl.pallas_call(
        body,
        grid=(N // blk_rows,),
        in_specs=[
            pl.BlockSpec((N, N_Z), lambda i: (0, 0)),
            pl.BlockSpec((blk_rows, N), lambda i: (i, 0)),
        ],
        out_specs=(
            pl.BlockSpec((blk_rows, N), lambda i: (i, 0)),
            pl.BlockSpec((1, 1), lambda i: (0, 0), memory_space=pltpu.SMEM),
        ),
        out_shape=(
            jax.ShapeDtypeStruct((N, N), jnp.float32),
            jax.ShapeDtypeStruct((1, 1), jnp.float32),
        ),
    )(z, adj)


def _spmm(edge_index, edge_weight, h):
    msgs = jnp.take(h, edge_index[0], axis=0) * edge_weight[:, None]
    return jax.ops.segment_sum(msgs, edge_index[1], num_segments=N)


def kernel(X, edge_index, edge_weight, adj_label, eps, W1, W_mean, W_logsig):
    xw1 = _matmul(X, W1, 1024)
    h1 = jax.nn.relu(_spmm(edge_index, edge_weight, xw1))
    wcat = jnp.concatenate([W_mean, W_logsig], axis=1)
    hcat = _matmul(h1, wcat, 1024)
    zcat = _spmm(edge_index, edge_weight, hcat)
    z_mean = zcat[:, :N_Z]
    z_log_sigma = zcat[:, N_Z:]
    z, kl = _reparam_kl(z_mean, z_log_sigma, eps)
    a, ce = _decoder_loss(z, adj_label)
    loss = (ce[0, 0] + kl[0, 0]).astype(jnp.float32)
    return (a, loss)
